# baseline (speedup 1.0000x reference)
import functools

import jax
import jax.numpy as jnp
from jax import lax
from jax.experimental import pallas as pl
from jax.experimental.pallas import tpu as pltpu
from jax.experimental.pallas import tpu_sc as plsc

_N = 10000
_E = 160000
_D = 256
_DH = 128
_DHID = 512
_NC = 2
_NS = 16
_CH = 128
_GRP = 8
_NGRP = 10
_CPT = _GRP * _NGRP
_NCHK = _NS * _CPT
_EPAD = _NCHK * _CH
_NPAD = 10240
_TRASH = 10000
_STRIPE = _NPAD // _NS
_AROWS = _NPAD // _NS
_P = 128
_HB = 8

_mesh = plsc.VectorSubcoreMesh(
    core_axis_name="c", subcore_axis_name="s", num_cores=_NC, num_subcores=_NS
)


def _sc_agg_body(x2, src2, dst2, agg_o, cnt_o,
                 gidx_v, dst_v, rows_v, hist_v, hbuf_v, cbuf_v,
                 acc_sh, cpart_sh, gsem, isem):
    c = lax.axis_index("c")
    s = lax.axis_index("s")
    zero16 = jnp.zeros((16,), jnp.float32)
    one16 = jnp.ones((16,), jnp.float32)

    def _zh(i, _):
        hist_v[pl.ds(i * 16, 16)] = zero16
        return 0

    lax.fori_loop(0, _NPAD // 16, _zh, 0)

    def _zr(i, _):
        def _zc(j, _):
            rows_v[i, pl.ds(j * 16, 16)] = zero16
            return 0

        lax.fori_loop(0, _DH // 16, _zc, 0)
        return 0

    lax.fori_loop(0, _P, _zr, 0)

    for p in range(_AROWS // _P):
        pltpu.sync_copy(
            rows_v.at[pl.ds(0, _P)],
            acc_sh.at[pl.ds(s * _AROWS + p * _P, _P)],
        )

    plsc.subcore_barrier()

    def _load_group(k, slot):
        r0 = s * _CPT + k * _GRP
        d1 = pltpu.async_copy(src2.at[pl.ds(r0, _GRP)], gidx_v.at[slot], isem)
        d2 = pltpu.async_copy(dst2.at[pl.ds(r0, _GRP)], dst_v.at[slot], isem)
        return d1, d2

    def _xform_group(slot):
        for u in range(_GRP):
            for j in range(_CH // 16):
                sl = pl.ds(j * 16, 16)
                gidx_v[slot, u, sl] = gidx_v[slot, u, sl] * 2 + c

    def _group(k, kslot, nslot, last):
        if not last:
            idescs = _load_group(k + 1, nslot)
        for u in range(_GRP):
            gd = pltpu.async_copy(
                x2.at[gidx_v.at[kslot, u]], rows_v, gsem)

            @pl.when(c == 0)
            def _():
                for j in range(_CH // 16):
                    d16 = dst_v[kslot, u, pl.ds(j * 16, 16)]
                    plsc.addupdate_scatter(hist_v, [d16], one16)

            if u == _GRP - 1 and not last:
                idescs[0].wait()
                idescs[1].wait()
                _xform_group(nslot)

            gd.wait()
            pltpu.sync_copy(
                rows_v, acc_sh.at[dst_v.at[kslot, u]], add=True)

    d1, d2 = _load_group(0, 0)
    d1.wait()
    d2.wait()
    _xform_group(0)

    def _pairbody(t, _):
        _group(t * 2, 0, 1, False)
        _group(t * 2 + 1, 1, 0, False)
        return 0

    lax.fori_loop(0, _NGRP // 2 - 1, _pairbody, 0)
    _group(_NGRP - 2, 0, 1, False)
    _group(_NGRP - 1, 1, 0, True)

    @pl.when(c == 0)
    def _():
        pltpu.sync_copy(hist_v, cpart_sh.at[s])

    plsc.subcore_barrier()

    @pl.when(c == 0)
    def _():
        for h in range(_NS // _HB):
            pltpu.sync_copy(
                cpart_sh.at[pl.ds(h * _HB, _HB), pl.ds(s * _STRIPE, _STRIPE)],
                hbuf_v,
            )

            def _red(j, _):
                sl = pl.ds(j * 16, 16)
                a = hbuf_v[0, sl]
                for t in range(1, _HB):
                    a = a + hbuf_v[t, sl]
                if h == 0:
                    cbuf_v[sl] = a
                else:
                    cbuf_v[sl] = cbuf_v[sl] + a
                return 0

            lax.fori_loop(0, _STRIPE // 16, _red, 0)
        pltpu.sync_copy(cbuf_v, cnt_o.at[pl.ds(s * _STRIPE, _STRIPE)])

    for p in range(_AROWS // _P):
        r0 = s * _AROWS + p * _P
        pltpu.sync_copy(acc_sh.at[pl.ds(r0, _P)], rows_v.at[pl.ds(0, _P)])
        pltpu.sync_copy(rows_v.at[pl.ds(0, _P)], agg_o.at[c, pl.ds(r0, _P)])


_sc_agg = functools.partial(
    pl.kernel,
    out_type=(
        jax.ShapeDtypeStruct((_NC, _NPAD, _DH), jnp.float32),
        jax.ShapeDtypeStruct((_NPAD,), jnp.float32),
    ),
    mesh=_mesh,
    scratch_types=[
        pltpu.VMEM((2, _GRP, _CH), jnp.int32),
        pltpu.VMEM((2, _GRP, _CH), jnp.int32),
        pltpu.VMEM((_CH, _DH), jnp.float32),
        pltpu.VMEM((_NPAD,), jnp.float32),
        pltpu.VMEM((_HB, _STRIPE), jnp.float32),
        pltpu.VMEM((_STRIPE,), jnp.float32),
        pltpu.VMEM_SHARED((_NPAD, _DH), jnp.float32),
        pltpu.VMEM_SHARED((_NS, _NPAD), jnp.float32),
        pltpu.SemaphoreType.DMA,
        pltpu.SemaphoreType.DMA,
    ],
    compiler_params=pltpu.CompilerParams(needs_layout_passes=False),
)(_sc_agg_body)


_BN = 1000


def _tc_dense_body(agg_ref, cnt_ref, x_ref, wn_ref, bn_ref, ws_ref,
                   w1_ref, b1_ref, w2_ref, b2_ref, o_ref):
    a0 = agg_ref[0]
    a1 = agg_ref[1]
    recip = 1.0 / jnp.maximum(cnt_ref[...], 1.0)
    xb = x_ref[...]
    wn = wn_ref[...]
    conv = (
        jnp.dot(a0 * recip, wn[:_DH], preferred_element_type=jnp.float32)
        + jnp.dot(a1 * recip, wn[_DH:], preferred_element_type=jnp.float32)
        + jnp.dot(xb, ws_ref[...], preferred_element_type=jnp.float32)
        + bn_ref[...]
    )
    h = jnp.maximum(conv, 0.0)
    z = xb + h
    hid = jnp.maximum(
        jnp.dot(z, w1_ref[...], preferred_element_type=jnp.float32) + b1_ref[...],
        0.0,
    )
    o_ref[...] = h + jnp.dot(hid, w2_ref[...], preferred_element_type=jnp.float32) + b2_ref[...]


def _tc_dense(agg, cnt, x, wn, bn, ws, w1, b1, w2, b2):
    return pl.pallas_call(
        _tc_dense_body,
        grid=(_N // _BN,),
        in_specs=[
            pl.BlockSpec((_NC, _BN, _DH), lambda i: (0, i, 0)),
            pl.BlockSpec((_BN, 1), lambda i: (i, 0)),
            pl.BlockSpec((_BN, _D), lambda i: (i, 0)),
            pl.BlockSpec((_D, _D), lambda i: (0, 0)),
            pl.BlockSpec((1, _D), lambda i: (0, 0)),
            pl.BlockSpec((_D, _D), lambda i: (0, 0)),
            pl.BlockSpec((_D, _DHID), lambda i: (0, 0)),
            pl.BlockSpec((1, _DHID), lambda i: (0, 0)),
            pl.BlockSpec((_DHID, _D), lambda i: (0, 0)),
            pl.BlockSpec((1, _D), lambda i: (0, 0)),
        ],
        out_specs=pl.BlockSpec((_BN, _D), lambda i: (i, 0)),
        out_shape=jax.ShapeDtypeStruct((_N, _D), jnp.float32),
    )(agg, cnt, x, wn, bn, ws, w1, b1, w2, b2)


def kernel(x, edge_index, W_neigh, b_neigh, W_self, W1, b1, W2, b2):
    src = edge_index[0].astype(jnp.int32)
    dst = edge_index[1].astype(jnp.int32)
    npad = _EPAD - _E
    src2 = jnp.concatenate(
        [src, jnp.zeros((npad,), jnp.int32)]).reshape(_NCHK, _CH)
    dst2 = jnp.concatenate(
        [dst, jnp.full((npad,), _TRASH, jnp.int32)]).reshape(_NCHK, _CH)
    x2 = x.reshape(2 * _N, _DH)
    agg_pad, cnt_pad = _sc_agg(x2, src2, dst2)
    agg = agg_pad[:, :_N, :]
    cnt = cnt_pad[:_N].reshape(_N, 1)
    return _tc_dense(
        agg, cnt, x, W_neigh, b_neigh.reshape(1, _D), W_self,
        W1, b1.reshape(1, _DHID), W2, b2.reshape(1, _D),
    )


# speedup vs baseline: 1.3300x; 1.3300x over previous
import functools

import jax
import jax.numpy as jnp
from jax import lax
from jax.experimental import pallas as pl
from jax.experimental.pallas import tpu as pltpu
from jax.experimental.pallas import tpu_sc as plsc

_N = 10000
_E = 160000
_D = 256
_DH = 128
_DHID = 512
_NC = 2
_NS = 16
_CH = 128
_NFULL = _E // _CH // _NS
_REM = _E // _CH - _NFULL * _NS
_NPAD = 10240
_STRIPE = _NPAD // _NS
_AROWS = _NPAD // _NS
_P = 128

_mesh = plsc.VectorSubcoreMesh(
    core_axis_name="c", subcore_axis_name="s", num_cores=_NC, num_subcores=_NS
)


def _sc_agg_body(x2, srcr, dstr, agg_o, cnt_o,
                 gidx_v, dst_v, rows_v, hist_v, hbuf_v, cbuf_v,
                 acc_sh, cpart_sh, sem):
    c = lax.axis_index("c")
    s = lax.axis_index("s")
    zero16 = jnp.zeros((16,), jnp.float32)
    one16 = jnp.ones((16,), jnp.float32)

    def _zh(i, _):
        hist_v[pl.ds(i * 16, 16)] = zero16
        return 0

    lax.fori_loop(0, _NPAD // 16, _zh, 0)

    def _zr(i, _):
        def _zc(j, _):
            rows_v[i, pl.ds(j * 16, 16)] = zero16
            return 0

        lax.fori_loop(0, _DH // 16, _zc, 0)
        return 0

    lax.fori_loop(0, _CH, _zr, 0)

    for p in range(_AROWS // _P):
        pltpu.sync_copy(
            rows_v.at[pl.ds(0, _P)],
            acc_sh.at[pl.ds(s * _AROWS + p * _P, _P)],
        )

    plsc.subcore_barrier()

    def _do_chunk(q):
        e0 = q * _CH
        pltpu.sync_copy(srcr.at[pl.ds(e0, _CH)], gidx_v)
        pltpu.sync_copy(dstr.at[pl.ds(e0, _CH)], dst_v)
        for j in range(_CH // 16):
            sl = pl.ds(j * 16, 16)
            gidx_v[sl] = gidx_v[sl] * 2 + c

        @pl.when(c == 0)
        def _():
            for j in range(_CH // 16):
                d16 = dst_v[pl.ds(j * 16, 16)]
                plsc.addupdate_scatter(hist_v, [d16], one16)

        pltpu.async_copy(x2.at[gidx_v], rows_v, sem).wait()
        pltpu.sync_copy(rows_v, acc_sh.at[dst_v], add=True)

    def _chunk_body(g, _):
        _do_chunk(g * _NS + s)
        return 0

    lax.fori_loop(0, _NFULL, _chunk_body, 0)

    @pl.when(s < _REM)
    def _():
        _do_chunk(_NFULL * _NS + s)

    @pl.when(c == 0)
    def _():
        pltpu.sync_copy(hist_v, cpart_sh.at[s])

    plsc.subcore_barrier()

    @pl.when(c == 0)
    def _():
        pltpu.sync_copy(cpart_sh.at[:, pl.ds(s * _STRIPE, _STRIPE)], hbuf_v)

        def _red(j, _):
            sl = pl.ds(j * 16, 16)
            a = hbuf_v[0, sl]
            for t in range(1, _NS):
                a = a + hbuf_v[t, sl]
            cbuf_v[sl] = a
            return 0

        lax.fori_loop(0, _STRIPE // 16, _red, 0)
        pltpu.sync_copy(cbuf_v, cnt_o.at[pl.ds(s * _STRIPE, _STRIPE)])

    for p in range(_AROWS // _P):
        r0 = s * _AROWS + p * _P
        pltpu.sync_copy(acc_sh.at[pl.ds(r0, _P)], rows_v.at[pl.ds(0, _P)])
        pltpu.sync_copy(rows_v.at[pl.ds(0, _P)], agg_o.at[c, pl.ds(r0, _P)])


_sc_agg = functools.partial(
    pl.kernel,
    out_type=(
        jax.ShapeDtypeStruct((_NC, _NPAD, _DH), jnp.float32),
        jax.ShapeDtypeStruct((_NPAD,), jnp.float32),
    ),
    mesh=_mesh,
    scratch_types=[
        pltpu.VMEM((_CH,), jnp.int32),
        pltpu.VMEM((_CH,), jnp.int32),
        pltpu.VMEM((_CH, _DH), jnp.float32),
        pltpu.VMEM((_NPAD,), jnp.float32),
        pltpu.VMEM((_NS, _STRIPE), jnp.float32),
        pltpu.VMEM((_STRIPE,), jnp.float32),
        pltpu.VMEM_SHARED((_NPAD, _DH), jnp.float32),
        pltpu.VMEM_SHARED((_NS, _NPAD), jnp.float32),
        pltpu.SemaphoreType.DMA,
    ],
    compiler_params=pltpu.CompilerParams(needs_layout_passes=False),
)(_sc_agg_body)


_BN = 1000


def _tc_dense_body(agg_ref, cnt_ref, x_ref, wn_ref, bn_ref, ws_ref,
                   w1_ref, b1_ref, w2_ref, b2_ref, o_ref):
    a0 = agg_ref[0]
    a1 = agg_ref[1]
    recip = 1.0 / jnp.maximum(cnt_ref[...], 1.0)
    xb = x_ref[...]
    wn = wn_ref[...]
    conv = (
        jnp.dot(a0 * recip, wn[:_DH], preferred_element_type=jnp.float32)
        + jnp.dot(a1 * recip, wn[_DH:], preferred_element_type=jnp.float32)
        + jnp.dot(xb, ws_ref[...], preferred_element_type=jnp.float32)
        + bn_ref[...]
    )
    h = jnp.maximum(conv, 0.0)
    z = xb + h
    hid = jnp.maximum(
        jnp.dot(z, w1_ref[...], preferred_element_type=jnp.float32) + b1_ref[...],
        0.0,
    )
    o_ref[...] = h + jnp.dot(hid, w2_ref[...], preferred_element_type=jnp.float32) + b2_ref[...]


def _tc_dense(agg, cnt, x, wn, bn, ws, w1, b1, w2, b2):
    return pl.pallas_call(
        _tc_dense_body,
        grid=(_N // _BN,),
        in_specs=[
            pl.BlockSpec((_NC, _BN, _DH), lambda i: (0, i, 0)),
            pl.BlockSpec((_BN, 1), lambda i: (i, 0)),
            pl.BlockSpec((_BN, _D), lambda i: (i, 0)),
            pl.BlockSpec((_D, _D), lambda i: (0, 0)),
            pl.BlockSpec((1, _D), lambda i: (0, 0)),
            pl.BlockSpec((_D, _D), lambda i: (0, 0)),
            pl.BlockSpec((_D, _DHID), lambda i: (0, 0)),
            pl.BlockSpec((1, _DHID), lambda i: (0, 0)),
            pl.BlockSpec((_DHID, _D), lambda i: (0, 0)),
            pl.BlockSpec((1, _D), lambda i: (0, 0)),
        ],
        out_specs=pl.BlockSpec((_BN, _D), lambda i: (i, 0)),
        out_shape=jax.ShapeDtypeStruct((_N, _D), jnp.float32),
    )(agg, cnt, x, wn, bn, ws, w1, b1, w2, b2)


def kernel(x, edge_index, W_neigh, b_neigh, W_self, W1, b1, W2, b2):
    src = edge_index[0].astype(jnp.int32)
    dst = edge_index[1].astype(jnp.int32)
    x2 = x.reshape(2 * _N, _DH)
    agg_pad, cnt_pad = _sc_agg(x2, src, dst)
    agg = agg_pad[:, :_N, :]
    cnt = cnt_pad[:_N].reshape(_N, 1)
    return _tc_dense(
        agg, cnt, x, W_neigh, b_neigh.reshape(1, _D), W_self,
        W1, b1.reshape(1, _DHID), W2, b2.reshape(1, _D),
    )


# speedup vs baseline: 1.8278x; 1.3744x over previous
import functools

import jax
import jax.numpy as jnp
from jax import lax
from jax.experimental import pallas as pl
from jax.experimental.pallas import tpu as pltpu
from jax.experimental.pallas import tpu_sc as plsc

_N = 10000
_E = 160000
_D = 256
_DH = 128
_DHID = 512
_NC = 2
_NS = 16
_CH = 128
_NFULL = _E // _CH // _NS
_REM = _E // _CH - _NFULL * _NS
_K = 6
_SL = 2
_NPAD = 10240
_AROWS = _NPAD // _NS
_P = 128
_HR = _NPAD // _DH
_HT = _HR // 8

_mesh = plsc.VectorSubcoreMesh(
    core_axis_name="c", subcore_axis_name="s", num_cores=_NC, num_subcores=_NS
)


def _sc_agg_body(x2, srcr, dstr, agg_o, cnt_o,
                 gidx_v, dst_v, rows_v, hist_v, hidx_v,
                 acc_sh, cnt_sh, gsem, ssem):
    c = lax.axis_index("c")
    s = lax.axis_index("s")
    zero16 = jnp.zeros((16,), jnp.float32)
    one16 = jnp.ones((16,), jnp.float32)
    iota16 = jnp.arange(16, dtype=jnp.int32)

    def _zh(i, _):
        def _zc(j, _):
            hist_v[i, pl.ds(j * 16, 16)] = zero16
            return 0

        lax.fori_loop(0, _DH // 16, _zc, 0)
        return 0

    lax.fori_loop(0, _HR, _zh, 0)

    for j in range(_HR // 16):
        hidx_v[pl.ds(j * 16, 16)] = iota16 + (j * 16)

    def _zr(i, _):
        def _zc(j, _):
            rows_v[0, i, pl.ds(j * 16, 16)] = zero16
            return 0

        lax.fori_loop(0, _DH // 16, _zc, 0)
        return 0

    lax.fori_loop(0, _CH, _zr, 0)

    for p in range(_AROWS // _P):
        pltpu.sync_copy(
            rows_v.at[0],
            acc_sh.at[pl.ds(s * _AROWS + p * _P, _P)],
        )

    @pl.when(jnp.logical_and(c == 0, s < _HT))
    def _():
        pltpu.sync_copy(
            rows_v.at[0, pl.ds(0, 8)], cnt_sh.at[pl.ds(s * 8, 8)])

    plsc.subcore_barrier()

    def _prep(q, slot):
        e0 = q * _CH
        pltpu.sync_copy(srcr.at[pl.ds(e0, _CH)], gidx_v.at[slot])
        pltpu.sync_copy(dstr.at[pl.ds(e0, _CH)], dst_v.at[slot])
        for j in range(_CH // 16):
            sl = pl.ds(j * 16, 16)
            gidx_v[slot, sl] = gidx_v[slot, sl] * 2 + c

        @pl.when(c == 0)
        def _():
            for j in range(_CH // 16):
                d16 = dst_v[slot, pl.ds(j * 16, 16)]
                r16 = lax.shift_right_logical(d16, 7)
                c16 = lax.bitwise_and(d16, 127)
                plsc.addupdate_scatter(hist_v, [r16, c16], one16)

    def _gather(slot):
        return pltpu.async_copy(x2.at[gidx_v.at[slot]], rows_v.at[slot], gsem)

    def _scatter(slot):
        return pltpu.async_copy(
            rows_v.at[slot], acc_sh.at[dst_v.at[slot]], ssem, add=True)

    def _pipe_chunks(qs):
        n = len(qs)
        gd = [None] * n
        sd = [None] * n
        for i in range(n):
            slot = i % _SL
            if i >= _SL:
                sd[i - _SL].wait()
            _prep(qs[i], slot)
            if i >= 1:
                gd[i - 1].wait()
                sd[i - 1] = _scatter((i - 1) % _SL)
            gd[i] = _gather(slot)
        gd[n - 1].wait()
        sd[n - 1] = _scatter((n - 1) % _SL)
        for i in range(max(0, n - _SL), n):
            sd[i].wait()

    def _chunk_body(g, _):
        _pipe_chunks([(g * _K + i) * _NS + s for i in range(_K)])
        return 0

    lax.fori_loop(0, _NFULL // _K, _chunk_body, 0)

    @pl.when(s < _REM)
    def _():
        q = _NFULL * _NS + s
        _prep(q, 0)
        pltpu.async_copy(x2.at[gidx_v.at[0]], rows_v.at[0], gsem).wait()
        pltpu.sync_copy(rows_v.at[0], acc_sh.at[dst_v.at[0]], add=True)

    @pl.when(c == 0)
    def _():
        pltpu.sync_copy(hist_v, cnt_sh.at[hidx_v], add=True)

    plsc.subcore_barrier()

    @pl.when(jnp.logical_and(c == 0, s < _HT))
    def _():
        pltpu.sync_copy(cnt_sh.at[pl.ds(s * 8, 8)], cnt_o.at[pl.ds(s * 8, 8)])

    for p in range(_AROWS // _P):
        r0 = s * _AROWS + p * _P
        pltpu.sync_copy(acc_sh.at[pl.ds(r0, _P)], rows_v.at[0])
        pltpu.sync_copy(rows_v.at[0], agg_o.at[c, pl.ds(r0, _P)])


_sc_agg = functools.partial(
    pl.kernel,
    out_type=(
        jax.ShapeDtypeStruct((_NC, _NPAD, _DH), jnp.float32),
        jax.ShapeDtypeStruct((_HR, _DH), jnp.float32),
    ),
    mesh=_mesh,
    scratch_types=[
        pltpu.VMEM((_SL, _CH), jnp.int32),
        pltpu.VMEM((_SL, _CH), jnp.int32),
        pltpu.VMEM((_SL, _CH, _DH), jnp.float32),
        pltpu.VMEM((_HR, _DH), jnp.float32),
        pltpu.VMEM((_HR,), jnp.int32),
        pltpu.VMEM_SHARED((_NPAD, _DH), jnp.float32),
        pltpu.VMEM_SHARED((_HR, _DH), jnp.float32),
        pltpu.SemaphoreType.DMA,
        pltpu.SemaphoreType.DMA,
    ],
    compiler_params=pltpu.CompilerParams(needs_layout_passes=False),
)(_sc_agg_body)


_BN = 1000


def _tc_dense_body(agg_ref, cnt_ref, x_ref, wn_ref, bn_ref, ws_ref,
                   w1_ref, b1_ref, w2_ref, b2_ref, o_ref):
    a0 = agg_ref[0]
    a1 = agg_ref[1]
    recip = 1.0 / jnp.maximum(cnt_ref[...], 1.0)
    xb = x_ref[...]
    wn = wn_ref[...]
    conv = (
        jnp.dot(a0 * recip, wn[:_DH], preferred_element_type=jnp.float32)
        + jnp.dot(a1 * recip, wn[_DH:], preferred_element_type=jnp.float32)
        + jnp.dot(xb, ws_ref[...], preferred_element_type=jnp.float32)
        + bn_ref[...]
    )
    h = jnp.maximum(conv, 0.0)
    z = xb + h
    hid = jnp.maximum(
        jnp.dot(z, w1_ref[...], preferred_element_type=jnp.float32) + b1_ref[...],
        0.0,
    )
    o_ref[...] = h + jnp.dot(hid, w2_ref[...], preferred_element_type=jnp.float32) + b2_ref[...]


def _tc_dense(agg, cnt, x, wn, bn, ws, w1, b1, w2, b2):
    return pl.pallas_call(
        _tc_dense_body,
        grid=(_N // _BN,),
        in_specs=[
            pl.BlockSpec((_NC, _BN, _DH), lambda i: (0, i, 0)),
            pl.BlockSpec((_BN, 1), lambda i: (i, 0)),
            pl.BlockSpec((_BN, _D), lambda i: (i, 0)),
            pl.BlockSpec((_D, _D), lambda i: (0, 0)),
            pl.BlockSpec((1, _D), lambda i: (0, 0)),
            pl.BlockSpec((_D, _D), lambda i: (0, 0)),
            pl.BlockSpec((_D, _DHID), lambda i: (0, 0)),
            pl.BlockSpec((1, _DHID), lambda i: (0, 0)),
            pl.BlockSpec((_DHID, _D), lambda i: (0, 0)),
            pl.BlockSpec((1, _D), lambda i: (0, 0)),
        ],
        out_specs=pl.BlockSpec((_BN, _D), lambda i: (i, 0)),
        out_shape=jax.ShapeDtypeStruct((_N, _D), jnp.float32),
    )(agg, cnt, x, wn, bn, ws, w1, b1, w2, b2)


def kernel(x, edge_index, W_neigh, b_neigh, W_self, W1, b1, W2, b2):
    src = edge_index[0].astype(jnp.int32)
    dst = edge_index[1].astype(jnp.int32)
    x2 = x.reshape(2 * _N, _DH)
    agg_pad, cnt_pad = _sc_agg(x2, src, dst)
    agg = agg_pad[:, :_N, :]
    cnt = cnt_pad.reshape(_NPAD)[:_N].reshape(_N, 1)
    return _tc_dense(
        agg, cnt, x, W_neigh, b_neigh.reshape(1, _D), W_self,
        W1, b1.reshape(1, _DHID), W2, b2.reshape(1, _D),
    )


# speedup vs baseline: 1.8327x; 1.0027x over previous
import functools

import jax
import jax.numpy as jnp
from jax import lax
from jax.experimental import pallas as pl
from jax.experimental.pallas import tpu as pltpu
from jax.experimental.pallas import tpu_sc as plsc

_N = 10000
_E = 160000
_D = 256
_DH = 128
_DHID = 512
_NC = 2
_NS = 16
_CH = 128
_NFULL = _E // _CH // _NS
_REM = _E // _CH - _NFULL * _NS
_K = 6
_SL = 2
_NPAD = 10240
_AROWS = _NPAD // _NS
_P = 128
_HR = _NPAD // _DH
_HT = _HR // 8

_mesh = plsc.VectorSubcoreMesh(
    core_axis_name="c", subcore_axis_name="s", num_cores=_NC, num_subcores=_NS
)


def _sc_agg_body(x2, srcr, dstr, agg_o, cnt_o,
                 gidx_v, dst_v, rows_v, hist_v, hidx_v,
                 acc_sh, cnt_sh, gsem, ssem):
    c = lax.axis_index("c")
    s = lax.axis_index("s")
    zero16 = jnp.zeros((16,), jnp.float32)
    one16 = jnp.ones((16,), jnp.float32)
    iota16 = jnp.arange(16, dtype=jnp.int32)

    def _zh(i, _):
        def _zc(j, _):
            hist_v[i, pl.ds(j * 16, 16)] = zero16
            return 0

        lax.fori_loop(0, _DH // 16, _zc, 0)
        return 0

    lax.fori_loop(0, _HR, _zh, 0)

    for j in range(_HR // 16):
        hidx_v[pl.ds(j * 16, 16)] = iota16 + (j * 16)

    def _zr(i, _):
        def _zc(j, _):
            rows_v[0, i, pl.ds(j * 16, 16)] = zero16
            return 0

        lax.fori_loop(0, _DH // 16, _zc, 0)
        return 0

    lax.fori_loop(0, _CH, _zr, 0)

    zdescs = [
        pltpu.async_copy(
            rows_v.at[0], acc_sh.at[pl.ds(s * _AROWS + p * _P, _P)], gsem)
        for p in range(_AROWS // _P)
    ]
    for d in zdescs:
        d.wait()

    @pl.when(jnp.logical_and(c == 0, s < _HT))
    def _():
        pltpu.sync_copy(
            rows_v.at[0, pl.ds(0, 8)], cnt_sh.at[pl.ds(s * 8, 8)])

    plsc.subcore_barrier()

    def _prep(q, slot):
        e0 = q * _CH
        pltpu.sync_copy(srcr.at[pl.ds(e0, _CH)], gidx_v.at[slot])
        pltpu.sync_copy(dstr.at[pl.ds(e0, _CH)], dst_v.at[slot])
        for j in range(_CH // 16):
            sl = pl.ds(j * 16, 16)
            gidx_v[slot, sl] = gidx_v[slot, sl] * 2 + c

        @pl.when(c == 0)
        def _():
            for j in range(_CH // 16):
                d16 = dst_v[slot, pl.ds(j * 16, 16)]
                r16 = lax.shift_right_logical(d16, 7)
                c16 = lax.bitwise_and(d16, 127)
                plsc.addupdate_scatter(hist_v, [r16, c16], one16)

    def _gather(slot):
        return pltpu.async_copy(x2.at[gidx_v.at[slot]], rows_v.at[slot], gsem)

    def _scatter(slot):
        return pltpu.async_copy(
            rows_v.at[slot], acc_sh.at[dst_v.at[slot]], ssem, add=True)

    def _pipe_chunks(qs):
        n = len(qs)
        gd = [None] * n
        sd = [None] * n
        for i in range(n):
            slot = i % _SL
            if i >= _SL:
                sd[i - _SL].wait()
            _prep(qs[i], slot)
            if i >= 1:
                gd[i - 1].wait()
                sd[i - 1] = _scatter((i - 1) % _SL)
            gd[i] = _gather(slot)
        gd[n - 1].wait()
        sd[n - 1] = _scatter((n - 1) % _SL)
        for i in range(max(0, n - _SL), n):
            sd[i].wait()

    def _chunk_body(g, _):
        _pipe_chunks([(g * _K + i) * _NS + s for i in range(_K)])
        return 0

    lax.fori_loop(0, _NFULL // _K, _chunk_body, 0)

    @pl.when(s < _REM)
    def _():
        q = _NFULL * _NS + s
        _prep(q, 0)
        pltpu.async_copy(x2.at[gidx_v.at[0]], rows_v.at[0], gsem).wait()
        pltpu.sync_copy(rows_v.at[0], acc_sh.at[dst_v.at[0]], add=True)

    @pl.when(c == 0)
    def _():
        pltpu.sync_copy(hist_v, cnt_sh.at[hidx_v], add=True)

    plsc.subcore_barrier()

    @pl.when(jnp.logical_and(c == 0, s < _HT))
    def _():
        pltpu.sync_copy(cnt_sh.at[pl.ds(s * 8, 8)], cnt_o.at[pl.ds(s * 8, 8)])

    odescs = []
    for p in range(_AROWS // _P):
        r0 = s * _AROWS + p * _P
        odescs.append(pltpu.async_copy(
            acc_sh.at[pl.ds(r0, _P)], agg_o.at[c, pl.ds(r0, _P)], gsem))
    for d in odescs:
        d.wait()


_sc_agg = functools.partial(
    pl.kernel,
    out_type=(
        jax.ShapeDtypeStruct((_NC, _NPAD, _DH), jnp.float32),
        jax.ShapeDtypeStruct((_HR, _DH), jnp.float32),
    ),
    mesh=_mesh,
    scratch_types=[
        pltpu.VMEM((_SL, _CH), jnp.int32),
        pltpu.VMEM((_SL, _CH), jnp.int32),
        pltpu.VMEM((_SL, _CH, _DH), jnp.float32),
        pltpu.VMEM((_HR, _DH), jnp.float32),
        pltpu.VMEM((_HR,), jnp.int32),
        pltpu.VMEM_SHARED((_NPAD, _DH), jnp.float32),
        pltpu.VMEM_SHARED((_HR, _DH), jnp.float32),
        pltpu.SemaphoreType.DMA,
        pltpu.SemaphoreType.DMA,
    ],
    compiler_params=pltpu.CompilerParams(needs_layout_passes=False),
)(_sc_agg_body)


_BN = 1000


def _tc_dense_body(agg_ref, cnt_ref, x_ref, wn_ref, bn_ref, ws_ref,
                   w1_ref, b1_ref, w2_ref, b2_ref, o_ref):
    a0 = agg_ref[0]
    a1 = agg_ref[1]
    recip = 1.0 / jnp.maximum(cnt_ref[...], 1.0)
    xb = x_ref[...]
    wn = wn_ref[...]
    conv = (
        jnp.dot(a0 * recip, wn[:_DH], preferred_element_type=jnp.float32)
        + jnp.dot(a1 * recip, wn[_DH:], preferred_element_type=jnp.float32)
        + jnp.dot(xb, ws_ref[...], preferred_element_type=jnp.float32)
        + bn_ref[...]
    )
    h = jnp.maximum(conv, 0.0)
    z = xb + h
    hid = jnp.maximum(
        jnp.dot(z, w1_ref[...], preferred_element_type=jnp.float32) + b1_ref[...],
        0.0,
    )
    o_ref[...] = h + jnp.dot(hid, w2_ref[...], preferred_element_type=jnp.float32) + b2_ref[...]


def _tc_dense(agg, cnt, x, wn, bn, ws, w1, b1, w2, b2):
    return pl.pallas_call(
        _tc_dense_body,
        grid=(_N // _BN,),
        in_specs=[
            pl.BlockSpec((_NC, _BN, _DH), lambda i: (0, i, 0)),
            pl.BlockSpec((_BN, 1), lambda i: (i, 0)),
            pl.BlockSpec((_BN, _D), lambda i: (i, 0)),
            pl.BlockSpec((_D, _D), lambda i: (0, 0)),
            pl.BlockSpec((1, _D), lambda i: (0, 0)),
            pl.BlockSpec((_D, _D), lambda i: (0, 0)),
            pl.BlockSpec((_D, _DHID), lambda i: (0, 0)),
            pl.BlockSpec((1, _DHID), lambda i: (0, 0)),
            pl.BlockSpec((_DHID, _D), lambda i: (0, 0)),
            pl.BlockSpec((1, _D), lambda i: (0, 0)),
        ],
        out_specs=pl.BlockSpec((_BN, _D), lambda i: (i, 0)),
        out_shape=jax.ShapeDtypeStruct((_N, _D), jnp.float32),
    )(agg, cnt, x, wn, bn, ws, w1, b1, w2, b2)


def kernel(x, edge_index, W_neigh, b_neigh, W_self, W1, b1, W2, b2):
    src = edge_index[0].astype(jnp.int32)
    dst = edge_index[1].astype(jnp.int32)
    x2 = x.reshape(2 * _N, _DH)
    agg_pad, cnt_pad = _sc_agg(x2, src, dst)
    agg = agg_pad[:, :_N, :]
    cnt = cnt_pad.reshape(_NPAD)[:_N].reshape(_N, 1)
    return _tc_dense(
        agg, cnt, x, W_neigh, b_neigh.reshape(1, _D), W_self,
        W1, b1.reshape(1, _DHID), W2, b2.reshape(1, _D),
    )
